# baseline (speedup 1.0000x reference)
import functools

import jax
import jax.numpy as jnp
from jax import lax
from jax.experimental import pallas as pl
from jax.experimental.pallas import tpu as pltpu
from jax.experimental.pallas import tpu_sc as plsc

_TAU = 1.5e-3
_R = 256


def _stage1(x, wt):
    B, D = x.shape
    K = wt.shape[1]
    BLK = 256

    def body(x_ref, wt_ref, idx_ref, gap_ref):
        xb = x_ref[...]
        wtb = wt_ref[...]
        s = lax.dot_general(
            xb, wtb, (((1,), (0,)), ((), ())),
            preferred_element_type=jnp.float32,
            precision=lax.Precision.HIGHEST,
        )
        wsq = jnp.sum(wtb * wtb, axis=0)
        d = wsq[None, :] - 2.0 * s
        m1 = jnp.min(d, axis=1)
        am = jnp.argmin(d, axis=1).astype(jnp.int32)
        kio = lax.broadcasted_iota(jnp.int32, d.shape, 1)
        masked = jnp.where(kio == am[:, None], jnp.inf, d)
        m2 = jnp.min(masked, axis=1)
        idx_ref[...] = am
        gap_ref[...] = m2 - m1

    return pl.pallas_call(
        body,
        grid=(B // BLK,),
        in_specs=[
            pl.BlockSpec((BLK, D), lambda i: (i, 0)),
            pl.BlockSpec((D, K), lambda i: (0, 0)),
        ],
        out_specs=[
            pl.BlockSpec((BLK,), lambda i: (i,)),
            pl.BlockSpec((BLK,), lambda i: (i,)),
        ],
        out_shape=[
            jax.ShapeDtypeStruct((B,), jnp.int32),
            jax.ShapeDtypeStruct((B,), jnp.float32),
        ],
    )(x, wt)


def _stage2(xta, wt):
    D, R = xta.shape
    K = wt.shape[1]
    KB = 128
    C = D // 8

    def body(xta_ref, wt_ref, out_ref, bv_ref, bi_ref):
        kk = pl.program_id(0)

        @pl.when(kk == 0)
        def _init():
            bv_ref[...] = jnp.full((1, R), jnp.inf, jnp.float32)
            bi_ref[...] = jnp.zeros((1, R), jnp.int32)

        for kg in range(KB // 8):

            def cbody(c, accs):
                xt = xta_ref[pl.ds(8 * c, 8), :]
                w8 = wt_ref[pl.ds(8 * c, 8), kg * 8:kg * 8 + 8]
                out = []
                for j in range(8):
                    df = xt - w8[:, j:j + 1]
                    sq = df * df
                    q04 = sq[0:4, :] + sq[4:8, :]
                    e = q04[0:2, :] + q04[2:4, :]
                    t8 = e[0:1, :] + e[1:2, :]
                    out.append(accs[j] + t8)
                return tuple(out)

            accs = lax.fori_loop(
                0, C, cbody,
                tuple(jnp.zeros((1, R), jnp.float32) for _ in range(8)))

            for j in range(8):
                kglob = kk * KB + kg * 8 + j
                dk = accs[j]
                better = dk < bv_ref[...]
                bv_ref[...] = jnp.where(better, dk, bv_ref[...])
                bi_ref[...] = jnp.where(better, kglob, bi_ref[...])

        @pl.when(kk == pl.num_programs(0) - 1)
        def _fin():
            out_ref[...] = jnp.broadcast_to(bi_ref[...], (8, R))

    return pl.pallas_call(
        body,
        grid=(K // KB,),
        in_specs=[
            pl.BlockSpec((D, R), lambda kk: (0, 0)),
            pl.BlockSpec((D, KB), lambda kk: (0, kk)),
        ],
        out_specs=pl.BlockSpec((8, R), lambda kk: (0, 0)),
        out_shape=jax.ShapeDtypeStruct((8, R), jnp.int32),
        scratch_shapes=[
            pltpu.VMEM((1, R), jnp.float32),
            pltpu.VMEM((1, R), jnp.int32),
        ],
    )(xta, wt)


def _gather_x_rows(x, rows):
    B, D = x.shape
    R = rows.shape[0]
    x3 = x.reshape(B, 1, D)

    def body(rows_ref, x_ref, out_ref):
        out_ref[...] = x_ref[...]

    xa = pl.pallas_call(
        body,
        grid_spec=pltpu.PrefetchScalarGridSpec(
            num_scalar_prefetch=1,
            grid=(R,),
            in_specs=[pl.BlockSpec((1, 1, D), lambda i, rows: (rows[i], 0, 0))],
            out_specs=pl.BlockSpec((1, 1, D), lambda i, rows: (i, 0, 0)),
        ),
        out_shape=jax.ShapeDtypeStruct((R, 1, D), jnp.float32),
    )(rows, x3)
    return xa.reshape(R, D)


def _sc_gather_rows(table, idx):
    K, Dp = table.shape
    B = idx.shape[0]
    info = plsc.get_sparse_core_info()
    NC, NS = info.num_cores, info.num_subcores
    NW = NC * NS
    b_per_w = B // NW
    mesh = plsc.VectorSubcoreMesh(core_axis_name="c", subcore_axis_name="s")

    @functools.partial(
        pl.kernel,
        mesh=mesh,
        out_type=jax.ShapeDtypeStruct((B, Dp), jnp.float32),
        scratch_types=[
            pltpu.VMEM((b_per_w,), jnp.int32),
            pltpu.VMEM((b_per_w, Dp), jnp.float32),
            pltpu.SemaphoreType.DMA,
        ],
    )
    def k(table_hbm, idx_hbm, out_hbm, idx_v, rows_v, sem):
        wid = lax.axis_index("s") * NC + lax.axis_index("c")
        base = wid * b_per_w
        pltpu.sync_copy(idx_hbm.at[pl.ds(base, b_per_w)], idx_v)
        pltpu.async_copy(table_hbm.at[idx_v], rows_v, sem).wait()
        pltpu.sync_copy(rows_v, out_hbm.at[pl.ds(base, b_per_w)])

    return k(table, idx)


def kernel(encoder_embedding, W):
    x, w = encoder_embedding, W
    B, D = x.shape
    K = w.shape[0]
    Dp = -(-D // 128) * 128

    wt = w.T
    wp = jnp.pad(w, ((0, 0), (0, Dp - D)))

    idx, gap = _stage1(x, wt)
    amb = jnp.nonzero(gap < _TAU, size=_R, fill_value=0)[0].astype(jnp.int32)
    xa = _gather_x_rows(x, amb)
    xta = xa.T
    fix = _stage2(xta, wt)[0]
    idx_full = idx.at[amb].set(fix)
    out = _sc_gather_rows(wp, idx_full)
    return out[:, :D]


# speedup vs baseline: 8.7521x; 8.7521x over previous
import functools

import jax
import jax.numpy as jnp
from jax import lax
from jax.experimental import pallas as pl
from jax.experimental.pallas import tpu as pltpu
from jax.experimental.pallas import tpu_sc as plsc

_TAU = 1.5e-3
_R = 256


def _stage1(x, wt):
    B, D = x.shape
    K = wt.shape[1]
    BLK = 256

    def body(x_ref, wt_ref, idx_ref, gap_ref):
        xb = x_ref[...]
        wtb = wt_ref[...]
        s = lax.dot_general(
            xb, wtb, (((1,), (0,)), ((), ())),
            preferred_element_type=jnp.float32,
            precision=lax.Precision.HIGHEST,
        )
        wsq = jnp.sum(wtb * wtb, axis=0)
        d = wsq[None, :] - 2.0 * s
        m1 = jnp.min(d, axis=1)
        am = jnp.argmin(d, axis=1).astype(jnp.int32)
        kio = lax.broadcasted_iota(jnp.int32, d.shape, 1)
        masked = jnp.where(kio == am[:, None], jnp.inf, d)
        m2 = jnp.min(masked, axis=1)
        idx_ref[...] = am
        gap_ref[...] = m2 - m1

    return pl.pallas_call(
        body,
        grid=(B // BLK,),
        in_specs=[
            pl.BlockSpec((BLK, D), lambda i: (i, 0)),
            pl.BlockSpec((D, K), lambda i: (0, 0)),
        ],
        out_specs=[
            pl.BlockSpec((BLK,), lambda i: (i,)),
            pl.BlockSpec((BLK,), lambda i: (i,)),
        ],
        out_shape=[
            jax.ShapeDtypeStruct((B,), jnp.int32),
            jax.ShapeDtypeStruct((B,), jnp.float32),
        ],
    )(x, wt)


def _stage2(xta, wt):
    D, R = xta.shape
    K = wt.shape[1]
    KB = 128
    C = D // 8

    def body(xta_ref, wt_ref, out_ref, bv_ref, bi_ref):
        kk = pl.program_id(0)

        @pl.when(kk == 0)
        def _init():
            bv_ref[...] = jnp.full((1, R), jnp.inf, jnp.float32)
            bi_ref[...] = jnp.zeros((1, R), jnp.int32)

        for kg in range(KB // 8):

            def cbody(c, accs):
                xt = xta_ref[pl.ds(8 * c, 8), :]
                w8 = wt_ref[pl.ds(8 * c, 8), kg * 8:kg * 8 + 8]
                out = []
                for j in range(8):
                    df = xt - w8[:, j:j + 1]
                    sq = df * df
                    q04 = sq[0:4, :] + sq[4:8, :]
                    e = q04[0:2, :] + q04[2:4, :]
                    t8 = e[0:1, :] + e[1:2, :]
                    out.append(accs[j] + t8)
                return tuple(out)

            accs = lax.fori_loop(
                0, C, cbody,
                tuple(jnp.zeros((1, R), jnp.float32) for _ in range(8)))

            for j in range(8):
                kglob = kk * KB + kg * 8 + j
                dk = accs[j]
                better = dk < bv_ref[...]
                bv_ref[...] = jnp.where(better, dk, bv_ref[...])
                bi_ref[...] = jnp.where(better, kglob, bi_ref[...])

        @pl.when(kk == pl.num_programs(0) - 1)
        def _fin():
            out_ref[...] = jnp.broadcast_to(bi_ref[...], (8, R))

    return pl.pallas_call(
        body,
        grid=(K // KB,),
        in_specs=[
            pl.BlockSpec((D, R), lambda kk: (0, 0)),
            pl.BlockSpec((D, KB), lambda kk: (0, kk)),
        ],
        out_specs=pl.BlockSpec((8, R), lambda kk: (0, 0)),
        out_shape=jax.ShapeDtypeStruct((8, R), jnp.int32),
        scratch_shapes=[
            pltpu.VMEM((1, R), jnp.float32),
            pltpu.VMEM((1, R), jnp.int32),
        ],
    )(xta, wt)


def _sc_gather_rows(table, idx):
    K, Dp = table.shape
    B = idx.shape[0]
    info = plsc.get_sparse_core_info()
    NC, NS = info.num_cores, info.num_subcores
    NW = NC * NS
    b_per_w = B // NW
    mesh = plsc.VectorSubcoreMesh(core_axis_name="c", subcore_axis_name="s")

    @functools.partial(
        pl.kernel,
        mesh=mesh,
        out_type=jax.ShapeDtypeStruct((B, Dp), jnp.float32),
        scratch_types=[
            pltpu.VMEM((b_per_w,), jnp.int32),
            pltpu.VMEM((b_per_w, Dp), jnp.float32),
            pltpu.SemaphoreType.DMA,
        ],
    )
    def k(table_hbm, idx_hbm, out_hbm, idx_v, rows_v, sem):
        wid = lax.axis_index("s") * NC + lax.axis_index("c")
        base = wid * b_per_w
        pltpu.sync_copy(idx_hbm.at[pl.ds(base, b_per_w)], idx_v)
        pltpu.async_copy(table_hbm.at[idx_v], rows_v, sem).wait()
        pltpu.sync_copy(rows_v, out_hbm.at[pl.ds(base, b_per_w)])

    return k(table, idx)


def kernel(encoder_embedding, W):
    x, w = encoder_embedding, W
    B, D = x.shape
    K = w.shape[0]
    Dp = -(-D // 128) * 128

    wt = w.T
    xp = jnp.pad(x, ((0, 0), (0, Dp - D)))
    wp = jnp.pad(w, ((0, 0), (0, Dp - D)))

    idx, gap = _stage1(x, wt)
    out = _sc_gather_rows(wp, idx)
    return out[:, :D]
